# baseline (speedup 1.0000x reference)
import jax
import jax.numpy as jnp
from jax.experimental import pallas as pl
from jax.experimental.pallas import tpu as pltpu

B, N, D, FH, FW, C = 8, 6, 41, 16, 44, 80
NX, NY = 200, 200
NPIX = FH * FW
PPB = N * D * NPIX
NROWS = NX * NY + 1
NCH = 16
CHUNK = PPB // NCH
UNROLL = 8


def _mat3(ref, bn, off):
    return tuple(tuple(ref[bn, off + 3 * i + j] for j in range(3))
                 for i in range(3))


def _geom_kernel(mats_ref, fr_ref, o_ref):
    bn = pl.program_id(0)
    ipr = _mat3(mats_ref, bn, 0)
    com = _mat3(mats_ref, bn, 9)
    pt = tuple(mats_ref[bn, 18 + k] for k in range(3))
    tr = tuple(mats_ref[bn, 21 + k] for k in range(3))

    def b16(t):
        return t.astype(jnp.bfloat16).astype(jnp.float32)

    def dot3(a, v0, v1, v2, extra=None):
        r = (a[2] * v2 + a[1] * v1) + a[0] * v0
        return r if extra is None else r + extra

    u = b16(fr_ref[0] - pt[0])
    v = b16(fr_ref[1] - pt[1])
    d = b16(fr_ref[2] - pt[2])
    q0 = dot3(ipr[0], u, v, d)
    q1 = dot3(ipr[1], u, v, d)
    q2 = dot3(ipr[2], u, v, d)
    r0 = b16(q0 * q2)
    r1 = b16(q1 * q2)
    q2b = b16(q2)
    x = dot3(com[0], r0, r1, q2b, tr[0])
    y = dot3(com[1], r0, r1, q2b, tr[1])
    z = dot3(com[2], r0, r1, q2b, tr[2])
    gx = ((x + 50.0) / 0.5).astype(jnp.int32)
    gy = ((y + 50.0) / 0.5).astype(jnp.int32)
    gz = ((z + 10.0) / 20.0).astype(jnp.int32)
    kept = ((gx >= 0) & (gx < NX) & (gy >= 0) & (gy < NY)
            & (gz >= 0) & (gz < 1))
    idx = gy * NX + gx
    o_ref[0] = jnp.where(kept, idx, NX * NY)


def _scatter_kernel(idx_ref, f_ref, o_ref, idx_smem, sem):
    ch = pl.program_id(1)

    @pl.when(ch == 0)
    def _():
        o_ref[...] = jnp.zeros_like(o_ref)

    cp = pltpu.make_async_copy(idx_ref.at[0, 0], idx_smem, sem)
    cp.start()
    cp.wait()

    def body(k, carry):
        base = k * UNROLL
        idxs = [idx_smem[base + t] for t in range(UNROLL)]
        feats = [f_ref[base + t, 0] for t in range(UNROLL)]
        sums = []
        for t in range(UNROLL):
            row = o_ref[idxs[t], 0]
            for s in range(t):
                row = jnp.where(idxs[t] == idxs[s], sums[s], row)
            sums.append(row + feats[t])
        for t in range(UNROLL):
            o_ref[idxs[t], 0] = sums[t]
        return carry

    jax.lax.fori_loop(0, CHUNK // UNROLL, body, 0)


def _compute_idx(rots, trans, intrins, post_rots, post_trans, frustum):
    inv_pr = jnp.linalg.inv(post_rots)
    combine = rots @ jnp.linalg.inv(intrins)
    bf = lambda t: jax.lax.optimization_barrier(
        t.astype(jnp.bfloat16)).astype(jnp.float32)
    mats = jnp.concatenate(
        [bf(inv_pr).reshape(B * N, 9),
         bf(combine).reshape(B * N, 9),
         post_trans.reshape(B * N, 3),
         trans.reshape(B * N, 3)], axis=1)
    fr = frustum.reshape(D, NPIX, 3).transpose(2, 0, 1)

    return pl.pallas_call(
        _geom_kernel,
        grid=(B * N,),
        in_specs=[
            pl.BlockSpec(memory_space=pltpu.SMEM),
            pl.BlockSpec((3, D, NPIX), lambda i: (0, 0, 0)),
        ],
        out_specs=pl.BlockSpec((1, D, NPIX), lambda i: (i, 0, 0)),
        out_shape=jax.ShapeDtypeStruct((B * N, D, NPIX), jnp.int32),
        compiler_params=pltpu.CompilerParams(
            dimension_semantics=("parallel",),
        ),
        name="lss_geometry",
    )(mats, fr)


def _pool(idx3, feats):
    return pl.pallas_call(
        _scatter_kernel,
        grid=(B, NCH),
        in_specs=[
            pl.BlockSpec((1, 1, CHUNK), lambda b, c: (b * NCH + c, 0, 0)),
            pl.BlockSpec((CHUNK, 1, C), lambda b, c: (b * NCH + c, 0, 0)),
        ],
        out_specs=pl.BlockSpec((NROWS, 1, C), lambda b, c: (b, 0, 0)),
        out_shape=jax.ShapeDtypeStruct((B * NROWS, 1, C), jnp.float32),
        scratch_shapes=[
            pltpu.SMEM((CHUNK,), jnp.int32),
            pltpu.SemaphoreType.DMA,
        ],
        compiler_params=pltpu.CompilerParams(
            dimension_semantics=("parallel", "arbitrary"),
            vmem_limit_bytes=56 * 1024 * 1024,
        ),
        name="lss_scatter_pool",
    )(idx3, feats)


def kernel(cam_features, rots, trans, intrins, post_rots, post_trans, frustum):
    idx = _compute_idx(rots, trans, intrins, post_rots, post_trans, frustum)
    idx3 = idx.reshape(B * NCH, 1, CHUNK)
    feats = cam_features.reshape(B * PPB, 1, C)
    pooled = _pool(idx3, feats)
    out = pooled.reshape(B, NROWS, C)[:, :NX * NY, :]
    return out.reshape(B, NY, NX, C).transpose(0, 3, 1, 2)


# speedup vs baseline: 1.1232x; 1.1232x over previous
import jax
import jax.numpy as jnp
from jax.experimental import pallas as pl
from jax.experimental.pallas import tpu as pltpu

B, N, D, FH, FW, C = 8, 6, 41, 16, 44, 80
NX, NY = 200, 200
NPIX = FH * FW
PPB = N * D * NPIX
NROWS = NX * NY + 1
NCH = 16
CHUNK = PPB // NCH
UNROLL = 4


def _mat3(ref, bn, off):
    return tuple(tuple(ref[bn, off + 3 * i + j] for j in range(3))
                 for i in range(3))


def _geom_kernel(mats_ref, fr_ref, o_ref):
    bn = pl.program_id(0)
    ipr = _mat3(mats_ref, bn, 0)
    com = _mat3(mats_ref, bn, 9)
    pt = tuple(mats_ref[bn, 18 + k] for k in range(3))
    tr = tuple(mats_ref[bn, 21 + k] for k in range(3))

    def b16(t):
        return t.astype(jnp.bfloat16).astype(jnp.float32)

    def dot3(a, v0, v1, v2, extra=None):
        r = (a[2] * v2 + a[1] * v1) + a[0] * v0
        return r if extra is None else r + extra

    u = b16(fr_ref[0] - pt[0])
    v = b16(fr_ref[1] - pt[1])
    d = b16(fr_ref[2] - pt[2])
    q0 = dot3(ipr[0], u, v, d)
    q1 = dot3(ipr[1], u, v, d)
    q2 = dot3(ipr[2], u, v, d)
    r0 = b16(q0 * q2)
    r1 = b16(q1 * q2)
    q2b = b16(q2)
    x = dot3(com[0], r0, r1, q2b, tr[0])
    y = dot3(com[1], r0, r1, q2b, tr[1])
    z = dot3(com[2], r0, r1, q2b, tr[2])
    gx = ((x + 50.0) / 0.5).astype(jnp.int32)
    gy = ((y + 50.0) / 0.5).astype(jnp.int32)
    gz = ((z + 10.0) / 20.0).astype(jnp.int32)
    kept = ((gx >= 0) & (gx < NX) & (gy >= 0) & (gy < NY)
            & (gz >= 0) & (gz < 1))
    idx = gy * NX + gx
    o_ref[0] = jnp.where(kept, idx, NX * NY)


def _scatter_kernel(idx_ref, f_ref, o_ref, idx_smem, sem):
    ch = pl.program_id(1)

    @pl.when(ch == 0)
    def _():
        o_ref[...] = jnp.zeros_like(o_ref)

    cp = pltpu.make_async_copy(idx_ref.at[0, 0], idx_smem, sem)
    cp.start()
    cp.wait()

    def body(k, carry):
        base = k * UNROLL
        idxs = [idx_smem[base + t] for t in range(UNROLL)]
        feats = [f_ref[base + t, 0] for t in range(UNROLL)]
        sums = []
        for t in range(UNROLL):
            row = o_ref[idxs[t], 0]
            for s in range(t):
                row = jnp.where(idxs[t] == idxs[s], sums[s], row)
            sums.append(row + feats[t])
        for t in range(UNROLL):
            o_ref[idxs[t], 0] = sums[t]
        return carry

    jax.lax.fori_loop(0, CHUNK // UNROLL, body, 0)


def _compute_idx(rots, trans, intrins, post_rots, post_trans, frustum):
    inv_pr = jnp.linalg.inv(post_rots)
    combine = rots @ jnp.linalg.inv(intrins)
    bf = lambda t: jax.lax.optimization_barrier(
        t.astype(jnp.bfloat16)).astype(jnp.float32)
    mats = jnp.concatenate(
        [bf(inv_pr).reshape(B * N, 9),
         bf(combine).reshape(B * N, 9),
         post_trans.reshape(B * N, 3),
         trans.reshape(B * N, 3)], axis=1)
    fr = frustum.reshape(D, NPIX, 3).transpose(2, 0, 1)

    return pl.pallas_call(
        _geom_kernel,
        grid=(B * N,),
        in_specs=[
            pl.BlockSpec(memory_space=pltpu.SMEM),
            pl.BlockSpec((3, D, NPIX), lambda i: (0, 0, 0)),
        ],
        out_specs=pl.BlockSpec((1, D, NPIX), lambda i: (i, 0, 0)),
        out_shape=jax.ShapeDtypeStruct((B * N, D, NPIX), jnp.int32),
        compiler_params=pltpu.CompilerParams(
            dimension_semantics=("parallel",),
        ),
        name="lss_geometry",
    )(mats, fr)


def _pool(idx3, feats):
    return pl.pallas_call(
        _scatter_kernel,
        grid=(B, NCH),
        in_specs=[
            pl.BlockSpec((1, 1, CHUNK), lambda b, c: (b * NCH + c, 0, 0)),
            pl.BlockSpec((CHUNK, 1, C), lambda b, c: (b * NCH + c, 0, 0)),
        ],
        out_specs=pl.BlockSpec((NROWS, 1, C), lambda b, c: (b, 0, 0)),
        out_shape=jax.ShapeDtypeStruct((B * NROWS, 1, C), jnp.float32),
        scratch_shapes=[
            pltpu.SMEM((CHUNK,), jnp.int32),
            pltpu.SemaphoreType.DMA,
        ],
        compiler_params=pltpu.CompilerParams(
            dimension_semantics=("parallel", "arbitrary"),
            vmem_limit_bytes=56 * 1024 * 1024,
        ),
        name="lss_scatter_pool",
    )(idx3, feats)


def kernel(cam_features, rots, trans, intrins, post_rots, post_trans, frustum):
    idx = _compute_idx(rots, trans, intrins, post_rots, post_trans, frustum)
    idx3 = idx.reshape(B * NCH, 1, CHUNK)
    feats = cam_features.reshape(B * PPB, 1, C)
    pooled = _pool(idx3, feats)
    out = pooled.reshape(B, NROWS, C)[:, :NX * NY, :]
    return out.reshape(B, NY, NX, C).transpose(0, 3, 1, 2)
